# baseline (speedup 1.0000x reference)
import functools

import jax
import jax.numpy as jnp
from jax import lax
from jax.experimental import pallas as pl
from jax.experimental.pallas import tpu as pltpu


def _rup(x, m):
    return (x + m - 1) // m * m


def _dncnn_pair_kernel(x_ref, w_ref, b_ref, wl_ref, o_ref,
                       buf_a, buf_b, *, H, W, C, F, Wp, BASE, M, PAD, PB):
    n_cr = w_ref.shape[0]
    L = 2 * F

    col = lax.rem(lax.broadcasted_iota(jnp.int32, (M, 1), 0), jnp.int32(Wp))
    valid = col < W

    BOT0 = (PAD + M - Wp) // 16 * 16
    for buf in (buf_a, buf_b):
        buf[pl.ds(0, PAD + Wp), :] = jnp.zeros((PAD + Wp, 9 * L),
                                               jnp.bfloat16)
        buf[pl.ds(BOT0, PB - BOT0), :] = jnp.zeros((PB - BOT0, 9 * L),
                                                   jnp.bfloat16)

    def store_nine(dst, a16):
        for kx in range(3):
            s = 16 - kx + 1 if kx else 17
            parts = [jnp.zeros((s, L), jnp.bfloat16), a16]
            if s + M < M + 16:
                parts.append(jnp.zeros((16 - s, L), jnp.bfloat16))
            vk = jnp.concatenate(parts, axis=0)[:M + 16]
            for ky in range(3):
                w0 = PAD + Wp - 16 - Wp * ky
                q = 3 * ky + kx
                dst[pl.ds(w0, M + 16), q * L:(q + 1) * L] = vk

    zlane = jnp.zeros((H, W, F - C), jnp.bfloat16)
    zrows = jnp.zeros((H, Wp - W, L), jnp.bfloat16)
    xcat = jnp.concatenate(
        [x_ref[0, 0].astype(jnp.bfloat16), zlane,
         x_ref[0, 1].astype(jnp.bfloat16), zlane], axis=2)
    a16_in = jnp.concatenate([xcat, zrows], axis=1).reshape(M, L)
    store_nine(buf_a, a16_in)

    def hidden_layer(src, dst, idx):
        acc = jnp.dot(src[pl.ds(PAD, M), :], w_ref[idx],
                      preferred_element_type=jnp.float32)
        acc = jnp.maximum(acc + b_ref[idx], 0.0)
        store_nine(dst, jnp.where(valid, acc, 0.0).astype(jnp.bfloat16))

    def layer_pair(p, c):
        hidden_layer(buf_a, buf_b, 2 * p)
        hidden_layer(buf_b, buf_a, 2 * p + 1)
        return c
    lax.fori_loop(0, n_cr // 2, layer_pair, 0)

    accl = jnp.dot(buf_a[pl.ds(PAD, M), :], wl_ref[...],
                   preferred_element_type=jnp.float32)
    s3 = accl.reshape(H, Wp, L)
    o_ref[0, 0] = x_ref[0, 0] - s3[:, :W, 0:C]
    o_ref[0, 1] = x_ref[0, 1] - s3[:, :W, F:F + C]


def _dncnn_forward(x_nchw, params):
    N, C, H, W = x_nchw.shape
    F = params[0][0].shape[-1]
    n_cr = len(params) - 1
    L = 2 * F

    Wp = _rup(W + 2, 16)
    BASE = Wp - W - 1
    M = H * Wp
    PAD = 96
    PB = _rup(PAD + M + Wp + 16, 16)
    assert M % 16 == 0 and PAD % 16 == 0 and PAD >= Wp + 16

    w_list = []
    for i, (w, scale, shift) in enumerate(params[:-1]):
        wf = w * scale
        if i == 0:
            wf = jnp.pad(wf, ((0, 0), (0, 0), (0, F - C), (0, 0)))
        w_list.append(wf.reshape(9, F, F))
    w9 = jnp.stack(w_list)
    w_bd = jnp.zeros((n_cr, 9, 2, F, 2, F), jnp.float32)
    w_bd = w_bd.at[:, :, 0, :, 0, :].set(w9).at[:, :, 1, :, 1, :].set(w9)
    w_bd = w_bd.reshape(n_cr, 9 * L, L).astype(jnp.bfloat16)

    b_all = jnp.stack([s for (_, _, s) in params[:-1]])
    b_all = jnp.concatenate([b_all, b_all], axis=1).reshape(n_cr, 1, L)

    wl, sl, _ = params[-1]
    wl9 = (wl * sl).reshape(9, F, C)
    wl_bd = jnp.zeros((9, 2, F, L), jnp.float32)
    wl_bd = wl_bd.at[:, 0, :, 0:C].set(wl9).at[:, 1, :, F:F + C].set(wl9)
    wl_bd = wl_bd.reshape(9 * L, L).astype(jnp.bfloat16)

    pad = N % 2
    x_nhwc = jnp.transpose(x_nchw, (0, 2, 3, 1))
    if pad:
        x_nhwc = jnp.concatenate([x_nhwc, x_nhwc[-1:]], axis=0)
    P = x_nhwc.shape[0] // 2
    x_pair = x_nhwc.reshape(P, 2, H, W, C)

    kfn = functools.partial(_dncnn_pair_kernel, H=H, W=W, C=C, F=F,
                            Wp=Wp, BASE=BASE, M=M, PAD=PAD, PB=PB)
    out = pl.pallas_call(
        kfn,
        out_shape=jax.ShapeDtypeStruct((P, 2, H, W, C), x_nchw.dtype),
        grid=(P,),
        in_specs=[
            pl.BlockSpec((1, 2, H, W, C), lambda p: (p, 0, 0, 0, 0)),
            pl.BlockSpec((n_cr, 9 * L, L), lambda p: (0, 0, 0)),
            pl.BlockSpec((n_cr, 1, L), lambda p: (0, 0, 0)),
            pl.BlockSpec((9 * L, L), lambda p: (0, 0)),
        ],
        out_specs=pl.BlockSpec((1, 2, H, W, C), lambda p: (p, 0, 0, 0, 0)),
        scratch_shapes=[pltpu.VMEM((PB, 9 * L), jnp.bfloat16),
                        pltpu.VMEM((PB, 9 * L), jnp.bfloat16)],
        compiler_params=pltpu.CompilerParams(
            dimension_semantics=("parallel",),
            vmem_limit_bytes=100 * 1024 * 1024),
    )(x_pair, w_bd, b_all, wl_bd)

    out = out.reshape(2 * P, H, W, C)
    if pad:
        out = out[:N]
    return jnp.transpose(out, (0, 3, 1, 2))


def kernel(x,
           w_0, scale_0, shift_0, w_1, scale_1, shift_1,
           w_2, scale_2, shift_2, w_3, scale_3, shift_3,
           w_4, scale_4, shift_4, w_5, scale_5, shift_5,
           w_6, scale_6, shift_6, w_7, scale_7, shift_7,
           w_8, scale_8, shift_8, w_9, scale_9, shift_9,
           w_10, scale_10, shift_10, w_11, scale_11, shift_11,
           w_12, scale_12, shift_12, w_13, scale_13, shift_13,
           w_14, scale_14, shift_14, w_15, scale_15, shift_15,
           w_16, scale_16, shift_16):
    params = [
        (w_0, scale_0, shift_0), (w_1, scale_1, shift_1),
        (w_2, scale_2, shift_2), (w_3, scale_3, shift_3),
        (w_4, scale_4, shift_4), (w_5, scale_5, shift_5),
        (w_6, scale_6, shift_6), (w_7, scale_7, shift_7),
        (w_8, scale_8, shift_8), (w_9, scale_9, shift_9),
        (w_10, scale_10, shift_10), (w_11, scale_11, shift_11),
        (w_12, scale_12, shift_12), (w_13, scale_13, shift_13),
        (w_14, scale_14, shift_14), (w_15, scale_15, shift_15),
        (w_16, scale_16, shift_16),
    ]
    return _dncnn_forward(x, params)


# speedup vs baseline: 1.2522x; 1.2522x over previous
import functools

import jax
import jax.numpy as jnp
from jax import lax
from jax.experimental import pallas as pl
from jax.experimental.pallas import tpu as pltpu


def _rup(x, m):
    return (x + m - 1) // m * m


def _dncnn_pair_kernel(x_ref, w_ref, b_ref, wl_ref, o_ref,
                       buf_a, buf_b, stage, *, H, W, C, F, Wp, BASE, M, S0,
                       FLAT):
    n_cr = w_ref.shape[0]
    L = 2 * F
    BOT0 = (BASE + (H + 1) * Wp) // 16 * 16

    col = lax.rem(lax.broadcasted_iota(jnp.int32, (M, 1), 0), jnp.int32(Wp))
    valid = col < W

    ztop = jnp.zeros((S0 + 16, 3 * L), jnp.bfloat16)
    zbot = jnp.zeros((FLAT - BOT0, 3 * L), jnp.bfloat16)
    for buf in (buf_a, buf_b):
        buf[pl.ds(0, S0 + 16), :] = ztop
        buf[pl.ds(BOT0, FLAT - BOT0), :] = zbot

    zc = jnp.zeros((1, L), jnp.bfloat16)
    zg = jnp.zeros((Wp - W - 1, L), jnp.bfloat16)
    zpad = jnp.zeros((W, F - C), jnp.bfloat16)
    SH = BASE + 1

    def _stage_row(h, c):
        xrow = jnp.concatenate(
            [x_ref[0, 0, h].astype(jnp.bfloat16), zpad,
             x_ref[0, 1, h].astype(jnp.bfloat16), zpad], axis=1)
        vrow = jnp.concatenate([zc, xrow, zg], axis=0)
        off = h * Wp + Wp
        for kx in range(3):
            vk = jnp.concatenate(
                [jnp.zeros((SH - kx, L), jnp.bfloat16),
                 vrow[:Wp - SH + kx]], axis=0)
            buf_a[pl.ds(off, Wp), kx * L:(kx + 1) * L] = vk
        return c
    lax.fori_loop(0, H, _stage_row, 0)

    def conv3x3(src, get_w):
        acc = None
        for ky in range(3):
            d = jnp.dot(src[pl.ds(BASE + 1 + ky * Wp, M), :], get_w(ky),
                        preferred_element_type=jnp.float32)
            acc = d if acc is None else acc + d
        return acc

    def hidden_layer(src, dst, idx):
        acc = conv3x3(src, lambda ky: w_ref[idx, ky])
        acc = jnp.maximum(acc + b_ref[idx], 0.0)
        a16 = jnp.where(valid, acc, 0.0).astype(jnp.bfloat16)
        for kx in range(3):
            parts = [jnp.zeros((17 - kx, L), jnp.bfloat16)]
            parts.append(a16[:M - 1 + kx] if kx < 2 else a16)
            if kx == 2:
                parts.append(jnp.zeros((1, L), jnp.bfloat16))
            dst[pl.ds(S0 - 16, M + 16), kx * L:(kx + 1) * L] = \
                jnp.concatenate(parts, axis=0)

    def layer_pair(p, c):
        hidden_layer(buf_a, buf_b, 2 * p)
        hidden_layer(buf_b, buf_a, 2 * p + 1)
        return c
    lax.fori_loop(0, n_cr // 2, layer_pair, 0)

    stage[...] = conv3x3(buf_a, lambda ky: wl_ref[ky])

    def _emit_row(h, c):
        off = h * Wp
        o_ref[0, 0, h] = x_ref[0, 0, h] - stage[pl.ds(off, W), 0:C]
        o_ref[0, 1, h] = x_ref[0, 1, h] - stage[pl.ds(off, W), F:F + C]
        return c
    lax.fori_loop(0, H, _emit_row, 0)


def _dncnn_forward(x_nchw, params):
    N, C, H, W = x_nchw.shape
    F = params[0][0].shape[-1]
    n_cr = len(params) - 1
    L = 2 * F

    Wp = _rup(W + 2, 16)
    BASE = Wp - W - 1
    M = H * Wp
    S0 = BASE + Wp + 1
    FLAT = _rup(BASE + (H + 2) * Wp + 2, 16)
    assert S0 % 16 == 0 and M % 16 == 0

    w_list = []
    for i, (w, scale, shift) in enumerate(params[:-1]):
        wf = w * scale
        if i == 0:
            wf = jnp.pad(wf, ((0, 0), (0, 0), (0, F - C), (0, 0)))
        w_list.append(wf.reshape(9, F, F))
    w9 = jnp.stack(w_list)
    w_bd = jnp.zeros((n_cr, 9, 2, F, 2, F), jnp.float32)
    w_bd = w_bd.at[:, :, 0, :, 0, :].set(w9).at[:, :, 1, :, 1, :].set(w9)
    w_bd = w_bd.reshape(n_cr, 3, 3 * L, L).astype(jnp.bfloat16)

    b_all = jnp.stack([s for (_, _, s) in params[:-1]])
    b_all = jnp.concatenate([b_all, b_all], axis=1).reshape(n_cr, 1, L)

    wl, sl, _ = params[-1]
    wl9 = (wl * sl).reshape(9, F, C)
    wl_bd = jnp.zeros((9, 2, F, L), jnp.float32)
    wl_bd = wl_bd.at[:, 0, :, 0:C].set(wl9).at[:, 1, :, F:F + C].set(wl9)
    wl_bd = wl_bd.reshape(3, 3 * L, L).astype(jnp.bfloat16)

    pad = N % 2
    x_nhwc = jnp.transpose(x_nchw, (0, 2, 3, 1))
    if pad:
        x_nhwc = jnp.concatenate([x_nhwc, x_nhwc[-1:]], axis=0)
    P = x_nhwc.shape[0] // 2
    x_pair = x_nhwc.reshape(P, 2, H, W, C)

    kfn = functools.partial(_dncnn_pair_kernel, H=H, W=W, C=C, F=F,
                            Wp=Wp, BASE=BASE, M=M, S0=S0, FLAT=FLAT)
    out = pl.pallas_call(
        kfn,
        out_shape=jax.ShapeDtypeStruct((P, 2, H, W, C), x_nchw.dtype),
        grid=(P,),
        in_specs=[
            pl.BlockSpec((1, 2, H, W, C), lambda p: (p, 0, 0, 0, 0)),
            pl.BlockSpec((n_cr, 3, 3 * L, L), lambda p: (0, 0, 0, 0)),
            pl.BlockSpec((n_cr, 1, L), lambda p: (0, 0, 0)),
            pl.BlockSpec((3, 3 * L, L), lambda p: (0, 0, 0)),
        ],
        out_specs=pl.BlockSpec((1, 2, H, W, C), lambda p: (p, 0, 0, 0, 0)),
        scratch_shapes=[pltpu.VMEM((FLAT, 3 * L), jnp.bfloat16),
                        pltpu.VMEM((FLAT, 3 * L), jnp.bfloat16),
                        pltpu.VMEM((M, L), jnp.float32)],
        compiler_params=pltpu.CompilerParams(
            dimension_semantics=("parallel",),
            vmem_limit_bytes=100 * 1024 * 1024),
    )(x_pair, w_bd, b_all, wl_bd)

    out = out.reshape(2 * P, H, W, C)
    if pad:
        out = out[:N]
    return jnp.transpose(out, (0, 3, 1, 2))


def kernel(x,
           w_0, scale_0, shift_0, w_1, scale_1, shift_1,
           w_2, scale_2, shift_2, w_3, scale_3, shift_3,
           w_4, scale_4, shift_4, w_5, scale_5, shift_5,
           w_6, scale_6, shift_6, w_7, scale_7, shift_7,
           w_8, scale_8, shift_8, w_9, scale_9, shift_9,
           w_10, scale_10, shift_10, w_11, scale_11, shift_11,
           w_12, scale_12, shift_12, w_13, scale_13, shift_13,
           w_14, scale_14, shift_14, w_15, scale_15, shift_15,
           w_16, scale_16, shift_16):
    params = [
        (w_0, scale_0, shift_0), (w_1, scale_1, shift_1),
        (w_2, scale_2, shift_2), (w_3, scale_3, shift_3),
        (w_4, scale_4, shift_4), (w_5, scale_5, shift_5),
        (w_6, scale_6, shift_6), (w_7, scale_7, shift_7),
        (w_8, scale_8, shift_8), (w_9, scale_9, shift_9),
        (w_10, scale_10, shift_10), (w_11, scale_11, shift_11),
        (w_12, scale_12, shift_12), (w_13, scale_13, shift_13),
        (w_14, scale_14, shift_14), (w_15, scale_15, shift_15),
        (w_16, scale_16, shift_16),
    ]
    return _dncnn_forward(x, params)
